# baseline (speedup 1.0000x reference)
import functools

import jax
import jax.numpy as jnp
from jax import lax
from jax.experimental import pallas as pl
from jax.experimental.pallas import tpu as pltpu
from jax.experimental.pallas import tpu_sc as plsc

NC = 2
NS = 16
NW = NC * NS
K = 40
R = 4
SL = 8
NPAR = 3
GB = K * 128 * 4
IB = SL * K * 4
G = 64


def _pad_edges(edge_index, n):
  e = edge_index.shape[1]
  gran = K * SL * NPAR
  epw = -(-e // (NW * gran)) * gran
  epad = NW * epw
  pad = epad - e
  row = edge_index[0]
  col = edge_index[1]
  trash = n + (jnp.arange(pad, dtype=jnp.int32) % 8)
  row_g = jnp.concatenate([row, jnp.zeros((pad,), jnp.int32)])
  col_s = jnp.concatenate([col, trash])
  row_s = jnp.concatenate([row, trash])
  nchunk = epw // K
  return (row_g.reshape(NW, nchunk, K), col_s.reshape(NW, nchunk, K),
          row_s.reshape(NW, nchunk, K), nchunk)


def _acc_rows(n):
  return -(-(n + 8) // K) * K


def _wsplit(n):
  wa = -(-(-(-n // NS)) // 8) * 8
  wl = n - (NS - 1) * wa
  assert wl > 0
  return wa, wl


def _zero_acc(acc, zsrc, s, na, d):
  nzc = na // K
  cpt = -(-nzc // NS)
  nv = d // 16

  def fill_zero(i, _):
    zsrc[i // nv, pl.ds((i % nv) * 16, 16)] = jnp.zeros((16,), jnp.float32)
    return 0
  lax.fori_loop(0, K * nv, fill_zero, 0)
  for j in range(cpt):
    idx = s * cpt + j

    @pl.when(idx < nzc)
    def _():
      pltpu.sync_copy(zsrc, acc.at[pl.ds(idx * K, K)])


def _writeout(acc, out_hbm, c, s, n):
  wa, wl = _wsplit(n)

  @pl.when(s < NS - 1)
  def _():
    pltpu.sync_copy(acc.at[pl.ds(s * wa, wa)],
                    out_hbm.at[c, pl.ds(s * wa, wa)])

  @pl.when(s == NS - 1)
  def _():
    pltpu.sync_copy(acc.at[pl.ds((NS - 1) * wa, wl)],
                    out_hbm.at[c, pl.ds((NS - 1) * wa, wl)])


def _make_deg_kernel(n, nchunk):
  na = _acc_rows(n)
  d = 128

  @functools.partial(
      pl.kernel,
      out_type=jax.ShapeDtypeStruct((NC, n, d), jnp.float32),
      mesh=plsc.VectorSubcoreMesh(core_axis_name="c", subcore_axis_name="s"),
      scratch_types=[
          pltpu.VMEM((nchunk, K), jnp.int32),
          pltpu.VMEM((K, d), jnp.float32),
          pltpu.VMEM_SHARED((na, d), jnp.float32),
          pltpu.SemaphoreType.DMA,
      ],
  )
  def deg_kernel(rows_hbm, out_hbm, rbuf, ones, acc, sem):
    c = lax.axis_index("c")
    s = lax.axis_index("s")
    wid = s * NC + c

    nv = d // 16
    _zero_acc(acc, ones, s, na, d)
    plsc.subcore_barrier()

    def fill_ones(i, _):
      ones[i // nv, pl.ds((i % nv) * 16, 16)] = jnp.ones((16,), jnp.float32)
      return 0
    lax.fori_loop(0, K * nv, fill_ones, 0)

    pltpu.sync_copy(rows_hbm.at[wid], rbuf)

    W = 8

    for j in range(W):
      pltpu.async_copy(ones, acc.at[rbuf.at[j]], sem, add=True)

    def body(j, _):
      pltpu.make_async_copy(ones, acc.at[rbuf.at[0]], sem).wait()
      pltpu.async_copy(ones, acc.at[rbuf.at[j]], sem, add=True)
      return 0
    lax.fori_loop(W, nchunk, body, 0)

    def drain(j, _):
      pltpu.make_async_copy(ones, acc.at[rbuf.at[0]], sem).wait()
      return 0
    lax.fori_loop(0, W, drain, 0)
    plsc.subcore_barrier()

    _writeout(acc, out_hbm, c, s, n)

  return deg_kernel


def _make_scatter_kernel(n, d, nchunk):
  na = _acc_rows(n)
  nslab = nchunk // SL
  nsup = nslab // NPAR
  assert nsup >= 3

  @functools.partial(
      pl.kernel,
      out_type=jax.ShapeDtypeStruct((NC, n, d), jnp.float32),
      mesh=plsc.VectorSubcoreMesh(core_axis_name="c", subcore_axis_name="s"),
      scratch_types=[
          [pltpu.VMEM((SL, K), jnp.int32) for _ in range(NPAR)],
          [pltpu.VMEM((SL, K), jnp.int32) for _ in range(NPAR)],
          [pltpu.VMEM((K, d), jnp.float32) for _ in range(R)],
          pltpu.VMEM_SHARED((na, d), jnp.float32),
          [pltpu.SemaphoreType.DMA for _ in range(R)],
          [pltpu.SemaphoreType.DMA for _ in range(R)],
          [pltpu.SemaphoreType.DMA for _ in range(NPAR)],
          [pltpu.SemaphoreType.DMA for _ in range(NPAR)],
      ],
  )
  def scatter_kernel(rows_hbm, cols_hbm, y_hbm, out_hbm,
                     rsl, csl, rows, acc, gsem, ssem, irsem, icsem):
    c = lax.axis_index("c")
    s = lax.axis_index("s")
    wid = s * NC + c

    _zero_acc(acc, rows[0], s, na, d)
    plsc.subcore_barrier()

    pltpu.sync_copy(rows_hbm.at[wid, pl.ds(0, SL)], rsl[0])
    pltpu.sync_copy(cols_hbm.at[wid, pl.ds(0, SL)], csl[0])
    for t in range(1, NPAR):
      pltpu.async_copy(rows_hbm.at[wid, pl.ds(t * SL, SL)], rsl[t], irsem[t])
      pltpu.async_copy(cols_hbm.at[wid, pl.ds(t * SL, SL)], csl[t], icsem[t])
    pltpu.async_copy(y_hbm.at[rsl[0].at[0]], rows[0], gsem[0])
    pltpu.async_copy(y_hbm.at[rsl[0].at[1]], rows[1], gsem[1])

    def chunk_ops(q, i, slab, j, mode):
      b = (q * SL + i) % R
      pltpu.make_async_copy(y_hbm.at[rsl[q].at[i]], rows[b], gsem[b]).wait()
      pltpu.async_copy(rows[b], acc.at[csl[q].at[i]], ssem[b], add=True)
      if i == 3 and (mode == 'mid' or (1 <= slab and slab + 2 < nslab)):
        qp = (q + 2) % NPAR
        pltpu.async_copy(rows_hbm.at[wid, pl.ds((slab + 2) * SL, SL)],
                         rsl[qp], irsem[qp])
        pltpu.async_copy(cols_hbm.at[wid, pl.ds((slab + 2) * SL, SL)],
                         csl[qp], icsem[qp])
      if mode != 'mid' and j + 2 >= nchunk:
        return
      if i == 6:
        pltpu.make_async_copy(rows_hbm.at[wid, pl.ds(0, SL)],
                              rsl[(q + 1) % NPAR],
                              irsem[(q + 1) % NPAR]).wait()
        pltpu.make_async_copy(cols_hbm.at[wid, pl.ds(0, SL)],
                              csl[(q + 1) % NPAR],
                              icsem[(q + 1) % NPAR]).wait()
      gi = i + 2
      bg = (q * SL + gi) % R
      if mode == 'mid' or j + 2 - R >= 0:
        pltpu.make_async_copy(rows[bg], acc.at[csl[q].at[i]],
                              ssem[bg]).wait()
      if gi < SL:
        idxref = rsl[q].at[gi]
      else:
        idxref = rsl[(q + 1) % NPAR].at[gi - SL]
      pltpu.async_copy(y_hbm.at[idxref], rows[bg], gsem[bg])

    for q in range(NPAR):
      for i in range(SL):
        chunk_ops(q, i, q, q * SL + i, 'pro')

    def outer(ss, _):
      for q in range(NPAR):
        slab = ss * NPAR + q
        for i in range(SL):
          chunk_ops(q, i, slab, slab * SL + i, 'mid')
      return 0
    lax.fori_loop(1, nsup - 1, outer, 0)

    for q in range(NPAR):
      slab = nslab - NPAR + q
      for i in range(SL):
        chunk_ops(q, i, slab, slab * SL + i, 'epi')

    for b in range(R):
      pltpu.make_async_copy(rows[b], acc.at[csl[0].at[0]], ssem[b]).wait()
    plsc.subcore_barrier()

    _writeout(acc, out_hbm, c, s, n)

  return scatter_kernel


def _tc_prescale(degp, x):
  n, d = x.shape

  def body(degp_ref, x_ref, dis_ref, y_ref):
    deg = degp_ref[0] + degp_ref[1]
    dis = jnp.where(deg > 0, lax.rsqrt(jnp.maximum(deg, 1e-30)), 0.0)
    dis_ref[...] = dis
    y_ref[...] = dis * x_ref[...]

  return pl.pallas_call(
      body,
      out_shape=[jax.ShapeDtypeStruct((n, 1), jnp.float32),
                 jax.ShapeDtypeStruct((n, d), jnp.float32)],
  )(degp, x)


def _tc_layer(h, p, dis, w0, w1, b, relu, want_y):
  n, d = h.shape
  hh = w0.shape[1]

  def body(h_ref, p_ref, dis_ref, w0_ref, w1_ref, b_ref, *outs):
    t = dis_ref[...] * (p_ref[0] + p_ref[1])
    z = (jnp.dot(h_ref[...], w0_ref[...], preferred_element_type=jnp.float32)
         - jnp.dot(t, w1_ref[...], preferred_element_type=jnp.float32)
         + b_ref[...][None, :])
    if relu:
      z = jnp.maximum(z, 0.0)
    outs[0][...] = z
    if want_y:
      outs[1][...] = dis_ref[...] * z

  out_shape = [jax.ShapeDtypeStruct((n, hh), jnp.float32)]
  if want_y:
    out_shape.append(jax.ShapeDtypeStruct((n, hh), jnp.float32))
  return pl.pallas_call(body, out_shape=out_shape)(h, p, dis, w0, w1, b)


def _tc_final(h, p, dis, w0, w1, b, batch2, wl, bl):
  n, d = h.shape
  hh = w0.shape[1]
  co = wl.shape[1]

  def body(h_ref, p_ref, dis_ref, w0_ref, w1_ref, b_ref, batch_ref,
           wl_ref, bl_ref, out_ref):
    t = dis_ref[...] * (p_ref[0] + p_ref[1])
    h3 = (jnp.dot(h_ref[...], w0_ref[...], preferred_element_type=jnp.float32)
          - jnp.dot(t, w1_ref[...], preferred_element_type=jnp.float32)
          + b_ref[...][None, :])
    seg = lax.broadcasted_iota(jnp.int32, (G, n), 0)
    m = (batch_ref[...] == seg).astype(jnp.float32)
    sums = jnp.dot(m, h3, preferred_element_type=jnp.float32)
    counts = jnp.sum(m, axis=1, keepdims=True)
    pooled = sums / jnp.maximum(counts, 1.0)
    out_ref[...] = (jnp.dot(pooled, wl_ref[...],
                            preferred_element_type=jnp.float32)
                    + bl_ref[...][None, :])

  return pl.pallas_call(
      body,
      out_shape=jax.ShapeDtypeStruct((G, co), jnp.float32),
  )(h, p, dis, w0, w1, b, batch2, wl, bl)


def kernel(x, edge_index, batch, W0_1, W1_1, b1, W0_2, W1_2, b2,
           W0_3, W1_3, b3, Wl, bl):
  n, d = x.shape
  row_g, col_s, row_s, nchunk = _pad_edges(edge_index, n)

  deg_kernel = _make_deg_kernel(n, nchunk)
  scat = _make_scatter_kernel(n, d, nchunk)

  degp = deg_kernel(row_s)
  dis, y0 = _tc_prescale(degp[:, :, 0:1], x)

  p1 = scat(row_g, col_s, y0)
  h1, y1 = _tc_layer(x, p1, dis, W0_1, W1_1, b1, relu=True, want_y=True)

  p2 = scat(row_g, col_s, y1)
  h2, y2 = _tc_layer(h1, p2, dis, W0_2, W1_2, b2, relu=True, want_y=True)

  p3 = scat(row_g, col_s, y2)
  batch2 = batch.reshape(1, n).astype(jnp.int32)
  return _tc_final(h2, p3, dis, W0_3, W1_3, b3, batch2, Wl, bl)


# speedup vs baseline: 1.9290x; 1.9290x over previous
import functools

import jax
import jax.numpy as jnp
from jax import lax
from jax.experimental import pallas as pl
from jax.experimental.pallas import tpu as pltpu
from jax.experimental.pallas import tpu_sc as plsc

NC = 2
NS = 16
NW = NC * NS
K = 128
G = 64


def _pad_edges(edge_index, n):
  e = edge_index.shape[1]
  epw = -(-e // (NW * K)) * K
  epad = NW * epw
  pad = epad - e
  row = edge_index[0]
  col = edge_index[1]
  trash = n + (jnp.arange(pad, dtype=jnp.int32) % 8)
  row_g = jnp.concatenate([row, jnp.zeros((pad,), jnp.int32)])
  col_s = jnp.concatenate([col, trash])
  row_s = jnp.concatenate([row, trash])
  nchunk = epw // K
  return (row_g.reshape(NW, nchunk, K), col_s.reshape(NW, nchunk, K),
          row_s.reshape(NW, nchunk, K), nchunk)


def _acc_rows(n):
  return -(-(n + 8) // K) * K


def _wsplit(n):
  wa = -(-(-(-n // NS)) // 8) * 8
  wl = n - (NS - 1) * wa
  assert wl > 0
  return wa, wl


def _zero_acc(acc, zsrc, s, na, d):
  nzc = na // K
  cpt = -(-nzc // NS)
  nv = d // 16

  def fill_zero(i, _):
    zsrc[i // nv, pl.ds((i % nv) * 16, 16)] = jnp.zeros((16,), jnp.float32)
    return 0
  lax.fori_loop(0, K * nv, fill_zero, 0)
  for j in range(cpt):
    idx = s * cpt + j

    @pl.when(idx < nzc)
    def _():
      pltpu.sync_copy(zsrc, acc.at[pl.ds(idx * K, K)])


def _writeout(acc, out_hbm, c, s, n):
  wa, wl = _wsplit(n)

  @pl.when(s < NS - 1)
  def _():
    pltpu.sync_copy(acc.at[pl.ds(s * wa, wa)],
                    out_hbm.at[c, pl.ds(s * wa, wa)])

  @pl.when(s == NS - 1)
  def _():
    pltpu.sync_copy(acc.at[pl.ds((NS - 1) * wa, wl)],
                    out_hbm.at[c, pl.ds((NS - 1) * wa, wl)])


def _make_deg_kernel(n, nchunk):
  na = _acc_rows(n)
  d = 128
  W = 8

  @functools.partial(
      pl.kernel,
      out_type=jax.ShapeDtypeStruct((NC, n, d), jnp.float32),
      mesh=plsc.VectorSubcoreMesh(core_axis_name="c", subcore_axis_name="s"),
      scratch_types=[
          pltpu.VMEM((nchunk, K), jnp.int32),
          pltpu.VMEM((K, d), jnp.float32),
          pltpu.VMEM_SHARED((na, d), jnp.float32),
          pltpu.SemaphoreType.DMA,
      ],
  )
  def deg_kernel(rows_hbm, out_hbm, rbuf, ones, acc, sem):
    c = lax.axis_index("c")
    s = lax.axis_index("s")
    wid = s * NC + c

    nv = d // 16
    _zero_acc(acc, ones, s, na, d)
    plsc.subcore_barrier()

    def fill_ones(i, _):
      ones[i // nv, pl.ds((i % nv) * 16, 16)] = jnp.ones((16,), jnp.float32)
      return 0
    lax.fori_loop(0, K * nv, fill_ones, 0)

    pltpu.sync_copy(rows_hbm.at[wid], rbuf)

    for j in range(W):
      pltpu.async_copy(ones, acc.at[rbuf.at[j]], sem, add=True)

    def body(j, _):
      pltpu.make_async_copy(ones, acc.at[rbuf.at[0]], sem).wait()
      pltpu.async_copy(ones, acc.at[rbuf.at[j]], sem, add=True)
      return 0
    lax.fori_loop(W, nchunk, body, 0)

    def drain(j, _):
      pltpu.make_async_copy(ones, acc.at[rbuf.at[0]], sem).wait()
      return 0
    lax.fori_loop(0, W, drain, 0)
    plsc.subcore_barrier()

    _writeout(acc, out_hbm, c, s, n)

  return deg_kernel


def _make_scatter_kernel(n, d, nchunk):
  na = _acc_rows(n)

  @functools.partial(
      pl.kernel,
      out_type=jax.ShapeDtypeStruct((NC, n, d), jnp.float32),
      mesh=plsc.VectorSubcoreMesh(core_axis_name="c", subcore_axis_name="s"),
      scratch_types=[
          pltpu.VMEM((nchunk, K), jnp.int32),
          pltpu.VMEM((nchunk, K), jnp.int32),
          pltpu.VMEM((K, d), jnp.float32),
          pltpu.VMEM_SHARED((na, d), jnp.float32),
          pltpu.SemaphoreType.DMA,
      ],
  )
  def scatter_kernel(rows_hbm, cols_hbm, y_hbm, out_hbm,
                     rbuf, cbuf, rows, acc, sem):
    c = lax.axis_index("c")
    s = lax.axis_index("s")
    wid = s * NC + c

    _zero_acc(acc, rows, s, na, d)
    plsc.subcore_barrier()

    pltpu.sync_copy(rows_hbm.at[wid], rbuf)
    pltpu.sync_copy(cols_hbm.at[wid], cbuf)

    def body(j, _):
      pltpu.async_copy(y_hbm.at[rbuf.at[j]], rows, sem).wait()
      pltpu.sync_copy(rows, acc.at[cbuf.at[j]], add=True)
      return 0
    lax.fori_loop(0, nchunk, body, 0)
    plsc.subcore_barrier()

    _writeout(acc, out_hbm, c, s, n)

  return scatter_kernel


def _tc_prescale(degp, x):
  n, d = x.shape

  def body(degp_ref, x_ref, dis_ref, y_ref):
    deg = degp_ref[0] + degp_ref[1]
    dis = jnp.where(deg > 0, lax.rsqrt(jnp.maximum(deg, 1e-30)), 0.0)
    dis_ref[...] = dis
    y_ref[...] = dis * x_ref[...]

  return pl.pallas_call(
      body,
      out_shape=[jax.ShapeDtypeStruct((n, 1), jnp.float32),
                 jax.ShapeDtypeStruct((n, d), jnp.float32)],
  )(degp, x)


def _tc_layer(h, p, dis, w0, w1, b, relu, want_y):
  n, d = h.shape
  hh = w0.shape[1]

  def body(h_ref, p_ref, dis_ref, w0_ref, w1_ref, b_ref, *outs):
    t = dis_ref[...] * (p_ref[0] + p_ref[1])
    z = (jnp.dot(h_ref[...], w0_ref[...], preferred_element_type=jnp.float32)
         - jnp.dot(t, w1_ref[...], preferred_element_type=jnp.float32)
         + b_ref[...][None, :])
    if relu:
      z = jnp.maximum(z, 0.0)
    outs[0][...] = z
    if want_y:
      outs[1][...] = dis_ref[...] * z

  out_shape = [jax.ShapeDtypeStruct((n, hh), jnp.float32)]
  if want_y:
    out_shape.append(jax.ShapeDtypeStruct((n, hh), jnp.float32))
  return pl.pallas_call(body, out_shape=out_shape)(h, p, dis, w0, w1, b)


def _tc_final(h, p, dis, w0, w1, b, batch2, wl, bl):
  n, d = h.shape
  hh = w0.shape[1]
  co = wl.shape[1]

  def body(h_ref, p_ref, dis_ref, w0_ref, w1_ref, b_ref, batch_ref,
           wl_ref, bl_ref, out_ref):
    t = dis_ref[...] * (p_ref[0] + p_ref[1])
    h3 = (jnp.dot(h_ref[...], w0_ref[...], preferred_element_type=jnp.float32)
          - jnp.dot(t, w1_ref[...], preferred_element_type=jnp.float32)
          + b_ref[...][None, :])
    seg = lax.broadcasted_iota(jnp.int32, (G, n), 0)
    m = (batch_ref[...] == seg).astype(jnp.float32)
    sums = jnp.dot(m, h3, preferred_element_type=jnp.float32)
    counts = jnp.sum(m, axis=1, keepdims=True)
    pooled = sums / jnp.maximum(counts, 1.0)
    out_ref[...] = (jnp.dot(pooled, wl_ref[...],
                            preferred_element_type=jnp.float32)
                    + bl_ref[...][None, :])

  return pl.pallas_call(
      body,
      out_shape=jax.ShapeDtypeStruct((G, co), jnp.float32),
  )(h, p, dis, w0, w1, b, batch2, wl, bl)


def kernel(x, edge_index, batch, W0_1, W1_1, b1, W0_2, W1_2, b2,
           W0_3, W1_3, b3, Wl, bl):
  n, d = x.shape
  row_g, col_s, row_s, nchunk = _pad_edges(edge_index, n)

  deg_kernel = _make_deg_kernel(n, nchunk)
  scat = _make_scatter_kernel(n, d, nchunk)

  degp = deg_kernel(row_s)
  dis, y0 = _tc_prescale(degp[:, :, 0:1], x)

  p1 = scat(row_g, col_s, y0)
  h1, y1 = _tc_layer(x, p1, dis, W0_1, W1_1, b1, relu=True, want_y=True)

  p2 = scat(row_g, col_s, y1)
  h2, y2 = _tc_layer(h1, p2, dis, W0_2, W1_2, b2, relu=True, want_y=True)

  p3 = scat(row_g, col_s, y2)
  batch2 = batch.reshape(1, n).astype(jnp.int32)
  return _tc_final(h2, p3, dis, W0_3, W1_3, b3, batch2, Wl, bl)
